# Initial kernel scaffold; baseline (speedup 1.0000x reference)
import jax
import jax.numpy as jnp
from jax.experimental import pallas as pl


def kernel(query, memory_keys, memory_values, temperature, ln_gamma, ln_beta):
    raise NotImplementedError("write your pallas kernel here")


# speedup vs baseline: 8.2186x; 8.2186x over previous
import functools

import jax
import jax.numpy as jnp
from jax import lax
from jax.experimental import pallas as pl
from jax.experimental.pallas import tpu as pltpu
from jax.experimental.pallas import tpu_sc as plsc

K = 32
CH = 128
NEG = -3.0e38
PAD = -1.0e30
BIG = 2147480000


def _simk_body(nm, bq, bm, m_real, q_ref, k_ref, t_ref, sims_ref, cm_ref,
               qn_ref, kn_ref):
    im = pl.program_id(0)
    iq = pl.program_id(1)

    @pl.when(im == 0)
    def _():
        q = q_ref[...]
        n = jnp.sqrt(jnp.sum(q * q, axis=1, keepdims=True))
        qn_ref[pl.ds(iq * bq, bq), :] = q / jnp.maximum(n, 1e-12) * jnp.abs(
            t_ref[0])

    @pl.when(iq == 0)
    def _():
        kk = k_ref[...]
        n = jnp.sqrt(jnp.sum(kk * kk, axis=1, keepdims=True))
        kn_ref[...] = kk / jnp.maximum(n, 1e-12)

    qn = qn_ref[pl.ds(iq * bq, bq), :]
    sims = lax.dot_general(qn, kn_ref[...], (((1,), (1,)), ((), ())),
                           preferred_element_type=jnp.float32)

    @pl.when(im == nm - 1)
    def _():
        col = im * bm + lax.broadcasted_iota(jnp.int32, (bq, bm), 1)
        sm = jnp.where(col < m_real, sims, PAD)
        sims_ref[...] = sm
        cm_ref[...] = jnp.max(sm.reshape(bq, bm // CH, CH),
                              axis=2).reshape(bq, 1, 1, bm // CH)

    @pl.when(im < nm - 1)
    def _():
        sims_ref[...] = sims
        cm_ref[...] = jnp.max(sims.reshape(bq, bm // CH, CH),
                              axis=2).reshape(bq, 1, 1, bm // CH)


def _similarity(query, keys, temperature):
    b, d = query.shape
    m = keys.shape[0]
    bq = min(1024, b)
    bm = 2048
    nm = (m + bm - 1) // bm
    if nm * bm - m < 2 * CH:
        nm += 1
    nq = b // bq
    mp = nm * bm
    nch = mp // CH
    kernel = pl.pallas_call(
        functools.partial(_simk_body, nm, bq, bm, m),
        grid=(nm, nq),
        in_specs=[
            pl.BlockSpec((bq, d), lambda im, iq: (iq, 0)),
            pl.BlockSpec((bm, d), lambda im, iq: (im, 0)),
            pl.BlockSpec(memory_space=pltpu.SMEM),
        ],
        out_specs=[
            pl.BlockSpec((bq, bm), lambda im, iq: (iq, im)),
            pl.BlockSpec((bq, 1, 1, bm // CH), lambda im, iq: (iq, im, 0, 0)),
        ],
        out_shape=[
            jax.ShapeDtypeStruct((b, mp), jnp.float32),
            jax.ShapeDtypeStruct((b, nm, 1, bm // CH), jnp.float32),
        ],
        scratch_shapes=[
            pltpu.VMEM((b, d), jnp.float32),
            pltpu.VMEM((bm, d), jnp.float32),
        ],
        compiler_params=pltpu.CompilerParams(
            dimension_semantics=("arbitrary", "arbitrary")),
    )
    sims, cm = kernel(query, keys, temperature.reshape(1))
    return sims, cm.reshape(b, nch), nch


def _chsel_body(nch, bq, cm_ref, cid_ref):
    iq = pl.program_id(0)
    cur = cm_ref[...]
    col = lax.broadcasted_iota(jnp.int32, (bq, nch), 1)
    kcol = lax.broadcasted_iota(jnp.int32, (bq, K), 1)
    rowg = iq * bq + lax.broadcasted_iota(jnp.int32, (bq, 1), 0)
    ids = jnp.zeros((bq, K), jnp.int32)
    for k in range(K):
        mx = jnp.max(cur, axis=1, keepdims=True)
        bi = jnp.min(jnp.where(cur == mx, col, BIG), axis=1, keepdims=True)
        ids = jnp.where(kcol == k, bi, ids)
        cur = jnp.where(col == bi, NEG, cur)
    cid_ref[...] = ids + rowg * nch


def _chunk_select(cm, nch):
    b = cm.shape[0]
    bq = min(1024, b)
    kernel = pl.pallas_call(
        functools.partial(_chsel_body, nch, bq),
        grid=(b // bq,),
        in_specs=[pl.BlockSpec((bq, nch), lambda i: (i, 0))],
        out_specs=pl.BlockSpec((bq, K), lambda i: (i, 0)),
        out_shape=jax.ShapeDtypeStruct((b, K), jnp.int32),
    )
    return kernel(cm)


def _scgather_body(blk, nblk, width, table_hbm, idx_hbm, out_hbm, idx_ref,
                   rows_ref, sem):
    wid = lax.axis_index("s") * 2 + lax.axis_index("c")
    base = wid * (nblk * blk)

    def it(j, c):
        off = base + j * blk
        pltpu.sync_copy(idx_hbm.at[pl.ds(off, blk)], idx_ref)
        pltpu.async_copy(table_hbm.at[idx_ref], rows_ref, sem).wait()
        pltpu.sync_copy(rows_ref, out_hbm.at[pl.ds(off, blk), :])
        return c

    lax.fori_loop(0, nblk, it, 0)


def _sc_gather(table, idx, width):
    n = idx.shape[0]
    blk = 128
    nblk = n // (32 * blk)
    mesh = plsc.VectorSubcoreMesh(core_axis_name="c", subcore_axis_name="s")
    kernel = pl.kernel(
        functools.partial(_scgather_body, blk, nblk, width),
        out_type=jax.ShapeDtypeStruct((n, width), jnp.float32),
        mesh=mesh,
        scratch_types=[
            pltpu.VMEM((blk,), jnp.int32),
            pltpu.VMEM((blk, width), jnp.float32),
            pltpu.SemaphoreType.DMA,
        ],
    )
    return kernel(table, idx)


def _fsel_body(nch, nc, bq, cand_ref, cid_ref, w_ref, idx_ref):
    iq = pl.program_id(0)
    rowg = iq * bq + lax.broadcasted_iota(jnp.int32, (bq, 1), 0)
    base = (cid_ref[...] - rowg * nch) * CH
    cidx = (base[:, :, None] +
            lax.broadcasted_iota(jnp.int32, (1, 1, CH), 2)).reshape(bq, nc)
    cur = cand_ref[...]
    kcol = lax.broadcasted_iota(jnp.int32, (bq, K), 1)
    scores = jnp.zeros((bq, K), jnp.float32)
    ids = jnp.zeros((bq, K), jnp.int32)
    for k in range(K):
        mx = jnp.max(cur, axis=1, keepdims=True)
        gi = jnp.min(jnp.where(cur == mx, cidx, BIG), axis=1, keepdims=True)
        scores = jnp.where(kcol == k, mx, scores)
        ids = jnp.where(kcol == k, gi, ids)
        cur = jnp.where(cidx == gi, NEG, cur)
    e = jnp.exp(scores - scores[:, 0:1])
    w_ref[...] = e / jnp.sum(e, axis=1, keepdims=True)
    idx_ref[...] = ids


def _final_select(cand, cid, nch):
    b = cid.shape[0]
    nc = K * CH
    bq = min(512, b)
    kernel = pl.pallas_call(
        functools.partial(_fsel_body, nch, nc, bq),
        grid=(b // bq,),
        in_specs=[
            pl.BlockSpec((bq, nc), lambda i: (i, 0)),
            pl.BlockSpec((bq, K), lambda i: (i, 0)),
        ],
        out_specs=[
            pl.BlockSpec((bq, K), lambda i: (i, 0)),
            pl.BlockSpec((bq, K), lambda i: (i, 0)),
        ],
        out_shape=[
            jax.ShapeDtypeStruct((b, K), jnp.float32),
            jax.ShapeDtypeStruct((b, K), jnp.int32),
        ],
    )
    return kernel(cand, cid)


def _wsum_body(rows_per_w, d, mv_hbm, idx_hbm, w_hbm, out_hbm, idx_sl, w_sl,
               vrows_ref, ret_sl, sem):
    wid = lax.axis_index("s") * 2 + lax.axis_index("c")
    dv = d // 16
    nk = rows_per_w * K
    base = wid * nk
    pltpu.sync_copy(idx_hbm.at[pl.ds(base, nk)], idx_sl)
    pltpu.sync_copy(w_hbm.at[pl.ds(base, nk)], w_sl.at[pl.ds(0, nk)])

    def row_body(i, _c):
        pltpu.async_copy(mv_hbm.at[idx_sl.at[pl.ds(i * K, K)]], vrows_ref,
                         sem).wait()

        def wsum(kk, acc):
            wk = w_sl[pl.ds(i * K + kk, 16)][0]
            return tuple(acc[j] + wk * vrows_ref[kk, pl.ds(j * 16, 16)]
                         for j in range(dv))

        acc = lax.fori_loop(
            0, K, wsum, tuple(jnp.zeros((16,), jnp.float32)
                              for _ in range(dv)))
        for j in range(dv):
            ret_sl[pl.ds(i * d + j * 16, 16)] = acc[j]
        return _c

    lax.fori_loop(0, rows_per_w, row_body, 0)
    pltpu.sync_copy(ret_sl, out_hbm.at[pl.ds(wid * rows_per_w * d,
                                             rows_per_w * d)])


def _sc_wsum(memory_values, idx, w):
    b = idx.shape[0]
    d = memory_values.shape[1]
    rows_per_w = b // 32
    mesh = plsc.VectorSubcoreMesh(core_axis_name="c", subcore_axis_name="s")
    kernel = pl.kernel(
        functools.partial(_wsum_body, rows_per_w, d),
        out_type=jax.ShapeDtypeStruct((b * d,), jnp.float32),
        mesh=mesh,
        scratch_types=[
            pltpu.VMEM((rows_per_w * K,), jnp.int32),
            pltpu.VMEM((rows_per_w * K + 16,), jnp.float32),
            pltpu.VMEM((K, d), jnp.float32),
            pltpu.VMEM((rows_per_w * d,), jnp.float32),
            pltpu.SemaphoreType.DMA,
        ],
    )
    out = kernel(memory_values, idx.reshape(-1), w.reshape(-1))
    return out.reshape(b, d)


def _ln_body(r_ref, q_ref, g_ref, b_ref, o_ref):
    x = r_ref[...] + q_ref[...]
    mu = jnp.mean(x, axis=1, keepdims=True)
    xc = x - mu
    var = jnp.mean(xc * xc, axis=1, keepdims=True)
    o_ref[...] = xc * lax.rsqrt(var + 1e-5) * g_ref[...] + b_ref[...]


def _layernorm(ret_raw, query, gamma, beta):
    b, d = query.shape
    bq = min(512, b)
    kernel = pl.pallas_call(
        _ln_body,
        grid=(b // bq,),
        in_specs=[
            pl.BlockSpec((bq, d), lambda i: (i, 0)),
            pl.BlockSpec((bq, d), lambda i: (i, 0)),
            pl.BlockSpec((1, d), lambda i: (0, 0)),
            pl.BlockSpec((1, d), lambda i: (0, 0)),
        ],
        out_specs=pl.BlockSpec((bq, d), lambda i: (i, 0)),
        out_shape=jax.ShapeDtypeStruct((b, d), jnp.float32),
    )
    return kernel(ret_raw, query, gamma.reshape(1, d), beta.reshape(1, d))


def kernel(query, memory_keys, memory_values, temperature, ln_gamma, ln_beta):
    b = query.shape[0]
    sims, cm, nch = _similarity(query, memory_keys, temperature)
    cid = _chunk_select(cm, nch)
    cand = _sc_gather(sims.reshape(-1, CH), cid.reshape(-1), CH)
    aw, idx = _final_select(cand.reshape(b, K * CH), cid, nch)
    ret_raw = _sc_wsum(memory_values, idx, aw)
    retrieved = _layernorm(ret_raw, query, ln_gamma, ln_beta)
    return retrieved, aw


# speedup vs baseline: 8.2273x; 1.0011x over previous
import functools

import jax
import jax.numpy as jnp
from jax import lax
from jax.experimental import pallas as pl
from jax.experimental.pallas import tpu as pltpu
from jax.experimental.pallas import tpu_sc as plsc

K = 32
CH = 128
NEG = -3.0e38
PAD = -1.0e30
BIG = 2147480000


def _simk_body(nm, bq, bm, m_real, q_ref, k_ref, t_ref, sims_ref, cm_ref,
               qn_ref, kn_ref):
    im = pl.program_id(0)
    iq = pl.program_id(1)

    @pl.when(im == 0)
    def _():
        q = q_ref[...]
        n = jnp.sqrt(jnp.sum(q * q, axis=1, keepdims=True))
        qn_ref[pl.ds(iq * bq, bq), :] = q / jnp.maximum(n, 1e-12) * jnp.abs(
            t_ref[0])

    @pl.when(iq == 0)
    def _():
        kk = k_ref[...]
        n = jnp.sqrt(jnp.sum(kk * kk, axis=1, keepdims=True))
        kn_ref[...] = kk / jnp.maximum(n, 1e-12)

    qn = qn_ref[pl.ds(iq * bq, bq), :].astype(jnp.bfloat16)
    sims = lax.dot_general(qn, kn_ref[...].astype(jnp.bfloat16),
                           (((1,), (1,)), ((), ())),
                           preferred_element_type=jnp.float32)

    @pl.when(im == nm - 1)
    def _():
        col = im * bm + lax.broadcasted_iota(jnp.int32, (bq, bm), 1)
        sm = jnp.where(col < m_real, sims, PAD)
        sims_ref[...] = sm
        cm_ref[...] = jnp.max(sm.reshape(bq, bm // CH, CH),
                              axis=2).reshape(bq, 1, 1, bm // CH)

    @pl.when(im < nm - 1)
    def _():
        sims_ref[...] = sims
        cm_ref[...] = jnp.max(sims.reshape(bq, bm // CH, CH),
                              axis=2).reshape(bq, 1, 1, bm // CH)


def _similarity(query, keys, temperature):
    b, d = query.shape
    m = keys.shape[0]
    bq = min(1024, b)
    bm = 2048
    nm = (m + bm - 1) // bm
    if nm * bm - m < 2 * CH:
        nm += 1
    nq = b // bq
    mp = nm * bm
    nch = mp // CH
    kernel = pl.pallas_call(
        functools.partial(_simk_body, nm, bq, bm, m),
        grid=(nm, nq),
        in_specs=[
            pl.BlockSpec((bq, d), lambda im, iq: (iq, 0)),
            pl.BlockSpec((bm, d), lambda im, iq: (im, 0)),
            pl.BlockSpec(memory_space=pltpu.SMEM),
        ],
        out_specs=[
            pl.BlockSpec((bq, bm), lambda im, iq: (iq, im)),
            pl.BlockSpec((bq, 1, 1, bm // CH), lambda im, iq: (iq, im, 0, 0)),
        ],
        out_shape=[
            jax.ShapeDtypeStruct((b, mp), jnp.float32),
            jax.ShapeDtypeStruct((b, nm, 1, bm // CH), jnp.float32),
        ],
        scratch_shapes=[
            pltpu.VMEM((b, d), jnp.float32),
            pltpu.VMEM((bm, d), jnp.float32),
        ],
        compiler_params=pltpu.CompilerParams(
            dimension_semantics=("arbitrary", "arbitrary")),
    )
    sims, cm = kernel(query, keys, temperature.reshape(1))
    return sims, cm.reshape(b, nch), nch


def _chsel_body(nch, bq, cm_ref, cid_ref):
    iq = pl.program_id(0)
    cur = cm_ref[...]
    col = lax.broadcasted_iota(jnp.int32, (bq, nch), 1)
    kcol = lax.broadcasted_iota(jnp.int32, (bq, K), 1)
    rowg = iq * bq + lax.broadcasted_iota(jnp.int32, (bq, 1), 0)
    ids = jnp.zeros((bq, K), jnp.int32)
    for k in range(K):
        mx = jnp.max(cur, axis=1, keepdims=True)
        bi = jnp.min(jnp.where(cur == mx, col, BIG), axis=1, keepdims=True)
        ids = jnp.where(kcol == k, bi, ids)
        cur = jnp.where(col == bi, NEG, cur)
    cid_ref[...] = ids + rowg * nch


def _chunk_select(cm, nch):
    b = cm.shape[0]
    bq = min(1024, b)
    kernel = pl.pallas_call(
        functools.partial(_chsel_body, nch, bq),
        grid=(b // bq,),
        in_specs=[pl.BlockSpec((bq, nch), lambda i: (i, 0))],
        out_specs=pl.BlockSpec((bq, K), lambda i: (i, 0)),
        out_shape=jax.ShapeDtypeStruct((b, K), jnp.int32),
    )
    return kernel(cm)


def _scgather_body(blk, nblk, width, table_hbm, idx_hbm, out_hbm, idx_ref,
                   rows_ref, sem):
    wid = lax.axis_index("s") * 2 + lax.axis_index("c")
    base = wid * (nblk * blk)

    def it(j, c):
        off = base + j * blk
        pltpu.sync_copy(idx_hbm.at[pl.ds(off, blk)], idx_ref)
        pltpu.async_copy(table_hbm.at[idx_ref], rows_ref, sem).wait()
        pltpu.sync_copy(rows_ref, out_hbm.at[pl.ds(off, blk), :])
        return c

    lax.fori_loop(0, nblk, it, 0)


def _sc_gather(table, idx, width):
    n = idx.shape[0]
    blk = 128
    nblk = n // (32 * blk)
    mesh = plsc.VectorSubcoreMesh(core_axis_name="c", subcore_axis_name="s")
    kernel = pl.kernel(
        functools.partial(_scgather_body, blk, nblk, width),
        out_type=jax.ShapeDtypeStruct((n, width), jnp.float32),
        mesh=mesh,
        scratch_types=[
            pltpu.VMEM((blk,), jnp.int32),
            pltpu.VMEM((blk, width), jnp.float32),
            pltpu.SemaphoreType.DMA,
        ],
    )
    return kernel(table, idx)


def _fsel_body(nch, nc, bq, cand_ref, cid_ref, w_ref, idx_ref):
    iq = pl.program_id(0)
    rowg = iq * bq + lax.broadcasted_iota(jnp.int32, (bq, 1), 0)
    base = (cid_ref[...] - rowg * nch) * CH
    cidx = (base[:, :, None] +
            lax.broadcasted_iota(jnp.int32, (1, 1, CH), 2)).reshape(bq, nc)
    cur = cand_ref[...]
    kcol = lax.broadcasted_iota(jnp.int32, (bq, K), 1)
    scores = jnp.zeros((bq, K), jnp.float32)
    ids = jnp.zeros((bq, K), jnp.int32)
    for k in range(K):
        mx = jnp.max(cur, axis=1, keepdims=True)
        gi = jnp.min(jnp.where(cur == mx, cidx, BIG), axis=1, keepdims=True)
        scores = jnp.where(kcol == k, mx, scores)
        ids = jnp.where(kcol == k, gi, ids)
        cur = jnp.where(cidx == gi, NEG, cur)
    e = jnp.exp(scores - scores[:, 0:1])
    w_ref[...] = e / jnp.sum(e, axis=1, keepdims=True)
    idx_ref[...] = ids


def _final_select(cand, cid, nch):
    b = cid.shape[0]
    nc = K * CH
    bq = min(512, b)
    kernel = pl.pallas_call(
        functools.partial(_fsel_body, nch, nc, bq),
        grid=(b // bq,),
        in_specs=[
            pl.BlockSpec((bq, nc), lambda i: (i, 0)),
            pl.BlockSpec((bq, K), lambda i: (i, 0)),
        ],
        out_specs=[
            pl.BlockSpec((bq, K), lambda i: (i, 0)),
            pl.BlockSpec((bq, K), lambda i: (i, 0)),
        ],
        out_shape=[
            jax.ShapeDtypeStruct((b, K), jnp.float32),
            jax.ShapeDtypeStruct((b, K), jnp.int32),
        ],
    )
    return kernel(cand, cid)


def _wsum_body(rows_per_w, d, mv_hbm, idx_hbm, w_hbm, out_hbm, idx_sl, w_sl,
               vrows_ref, ret_sl, sem):
    wid = lax.axis_index("s") * 2 + lax.axis_index("c")
    dv = d // 16
    nk = rows_per_w * K
    base = wid * nk
    pltpu.sync_copy(idx_hbm.at[pl.ds(base, nk)], idx_sl)
    pltpu.sync_copy(w_hbm.at[pl.ds(base, nk)], w_sl.at[pl.ds(0, nk)])

    def row_body(i, _c):
        pltpu.async_copy(mv_hbm.at[idx_sl.at[pl.ds(i * K, K)]], vrows_ref,
                         sem).wait()

        def wsum(kk, acc):
            wk = w_sl[pl.ds(i * K + kk, 16)][0]
            return tuple(acc[j] + wk * vrows_ref[kk, pl.ds(j * 16, 16)]
                         for j in range(dv))

        acc = lax.fori_loop(
            0, K, wsum, tuple(jnp.zeros((16,), jnp.float32)
                              for _ in range(dv)))
        for j in range(dv):
            ret_sl[pl.ds(i * d + j * 16, 16)] = acc[j]
        return _c

    lax.fori_loop(0, rows_per_w, row_body, 0)
    pltpu.sync_copy(ret_sl, out_hbm.at[pl.ds(wid * rows_per_w * d,
                                             rows_per_w * d)])


def _sc_wsum(memory_values, idx, w):
    b = idx.shape[0]
    d = memory_values.shape[1]
    rows_per_w = b // 32
    mesh = plsc.VectorSubcoreMesh(core_axis_name="c", subcore_axis_name="s")
    kernel = pl.kernel(
        functools.partial(_wsum_body, rows_per_w, d),
        out_type=jax.ShapeDtypeStruct((b * d,), jnp.float32),
        mesh=mesh,
        scratch_types=[
            pltpu.VMEM((rows_per_w * K,), jnp.int32),
            pltpu.VMEM((rows_per_w * K + 16,), jnp.float32),
            pltpu.VMEM((K, d), jnp.float32),
            pltpu.VMEM((rows_per_w * d,), jnp.float32),
            pltpu.SemaphoreType.DMA,
        ],
    )
    out = kernel(memory_values, idx.reshape(-1), w.reshape(-1))
    return out.reshape(b, d)


def _ln_body(r_ref, q_ref, g_ref, b_ref, o_ref):
    x = r_ref[...] + q_ref[...]
    mu = jnp.mean(x, axis=1, keepdims=True)
    xc = x - mu
    var = jnp.mean(xc * xc, axis=1, keepdims=True)
    o_ref[...] = xc * lax.rsqrt(var + 1e-5) * g_ref[...] + b_ref[...]


def _layernorm(ret_raw, query, gamma, beta):
    b, d = query.shape
    bq = min(512, b)
    kernel = pl.pallas_call(
        _ln_body,
        grid=(b // bq,),
        in_specs=[
            pl.BlockSpec((bq, d), lambda i: (i, 0)),
            pl.BlockSpec((bq, d), lambda i: (i, 0)),
            pl.BlockSpec((1, d), lambda i: (0, 0)),
            pl.BlockSpec((1, d), lambda i: (0, 0)),
        ],
        out_specs=pl.BlockSpec((bq, d), lambda i: (i, 0)),
        out_shape=jax.ShapeDtypeStruct((b, d), jnp.float32),
    )
    return kernel(ret_raw, query, gamma.reshape(1, d), beta.reshape(1, d))


def kernel(query, memory_keys, memory_values, temperature, ln_gamma, ln_beta):
    b = query.shape[0]
    sims, cm, nch = _similarity(query, memory_keys, temperature)
    cid = _chunk_select(cm, nch)
    cand = _sc_gather(sims.reshape(-1, CH), cid.reshape(-1), CH)
    aw, idx = _final_select(cand.reshape(b, K * CH), cid, nch)
    ret_raw = _sc_wsum(memory_values, idx, aw)
    retrieved = _layernorm(ret_raw, query, ln_gamma, ln_beta)
    return retrieved, aw


# speedup vs baseline: 8.2289x; 1.0002x over previous
import functools

import jax
import jax.numpy as jnp
from jax import lax
from jax.experimental import pallas as pl
from jax.experimental.pallas import tpu as pltpu
from jax.experimental.pallas import tpu_sc as plsc

K = 32
CH = 128
NEG = -3.0e38
PAD = -1.0e30
BIG = 2147480000


def _simk_body(nm, bq, bm, m_real, q_ref, k_ref, t_ref, sims_ref, cm_ref,
               qn_ref, kn_ref):
    im = pl.program_id(0)
    iq = pl.program_id(1)

    @pl.when(im == 0)
    def _():
        q = q_ref[...]
        n = jnp.sqrt(jnp.sum(q * q, axis=1, keepdims=True))
        qn_ref[pl.ds(iq * bq, bq), :] = q / jnp.maximum(n, 1e-12) * jnp.abs(
            t_ref[0])

    @pl.when(iq == 0)
    def _():
        kk = k_ref[...]
        n = jnp.sqrt(jnp.sum(kk * kk, axis=1, keepdims=True))
        kn_ref[...] = kk / jnp.maximum(n, 1e-12)

    qn = qn_ref[pl.ds(iq * bq, bq), :].astype(jnp.bfloat16)
    sims = lax.dot_general(qn, kn_ref[...].astype(jnp.bfloat16),
                           (((1,), (1,)), ((), ())),
                           preferred_element_type=jnp.float32)

    @pl.when(im == nm - 1)
    def _():
        col = im * bm + lax.broadcasted_iota(jnp.int32, (bq, bm), 1)
        sm = jnp.where(col < m_real, sims, PAD)
        sims_ref[...] = sm
        cm_ref[...] = jnp.max(sm.reshape(bq, bm // CH, CH),
                              axis=2).reshape(bq, 1, 1, bm // CH)

    @pl.when(im < nm - 1)
    def _():
        sims_ref[...] = sims
        cm_ref[...] = jnp.max(sims.reshape(bq, bm // CH, CH),
                              axis=2).reshape(bq, 1, 1, bm // CH)


def _similarity(query, keys, temperature):
    b, d = query.shape
    m = keys.shape[0]
    bq = min(1024, b)
    bm = 2048
    nm = (m + bm - 1) // bm
    if nm * bm - m < 2 * CH:
        nm += 1
    nq = b // bq
    mp = nm * bm
    nch = mp // CH
    kernel = pl.pallas_call(
        functools.partial(_simk_body, nm, bq, bm, m),
        grid=(nm, nq),
        in_specs=[
            pl.BlockSpec((bq, d), lambda im, iq: (iq, 0)),
            pl.BlockSpec((bm, d), lambda im, iq: (im, 0)),
            pl.BlockSpec(memory_space=pltpu.SMEM),
        ],
        out_specs=[
            pl.BlockSpec((bq, bm), lambda im, iq: (iq, im)),
            pl.BlockSpec((bq, 1, 1, bm // CH), lambda im, iq: (iq, im, 0, 0)),
        ],
        out_shape=[
            jax.ShapeDtypeStruct((b, mp), jnp.float32),
            jax.ShapeDtypeStruct((b, nm, 1, bm // CH), jnp.float32),
        ],
        scratch_shapes=[
            pltpu.VMEM((b, d), jnp.float32),
            pltpu.VMEM((bm, d), jnp.float32),
        ],
        compiler_params=pltpu.CompilerParams(
            dimension_semantics=("arbitrary", "arbitrary")),
    )
    sims, cm = kernel(query, keys, temperature.reshape(1))
    return sims, cm.reshape(b, nch), nch


def _chsel_body(nch, bq, cm_ref, cid_ref):
    iq = pl.program_id(0)
    cur = cm_ref[...]
    col = lax.broadcasted_iota(jnp.int32, (bq, nch), 1)
    kcol = lax.broadcasted_iota(jnp.int32, (bq, K), 1)
    rowg = iq * bq + lax.broadcasted_iota(jnp.int32, (bq, 1), 0)
    ids = jnp.zeros((bq, K), jnp.int32)
    for k in range(K):
        mx = jnp.max(cur, axis=1, keepdims=True)
        bi = jnp.min(jnp.where(cur == mx, col, BIG), axis=1, keepdims=True)
        ids = jnp.where(kcol == k, bi, ids)
        cur = jnp.where(col == bi, NEG, cur)
    cid_ref[...] = ids + rowg * nch


def _chunk_select(cm, nch):
    b = cm.shape[0]
    bq = min(1024, b)
    kernel = pl.pallas_call(
        functools.partial(_chsel_body, nch, bq),
        grid=(b // bq,),
        in_specs=[pl.BlockSpec((bq, nch), lambda i: (i, 0))],
        out_specs=pl.BlockSpec((bq, K), lambda i: (i, 0)),
        out_shape=jax.ShapeDtypeStruct((b, K), jnp.int32),
    )
    return kernel(cm)


def _scgather_body(blk, nblk, width, table_hbm, idx_hbm, out_hbm, idx_ref,
                   rows_ref, sem):
    wid = lax.axis_index("s") * 2 + lax.axis_index("c")
    base = wid * (nblk * blk)

    def it(j, c):
        off = base + j * blk
        pltpu.sync_copy(idx_hbm.at[pl.ds(off, blk)], idx_ref)
        pltpu.async_copy(table_hbm.at[idx_ref], rows_ref, sem).wait()
        pltpu.sync_copy(rows_ref, out_hbm.at[pl.ds(off, blk), :])
        return c

    lax.fori_loop(0, nblk, it, 0)


def _sc_gather(table, idx, width):
    n = idx.shape[0]
    blk = 128
    nblk = n // (32 * blk)
    mesh = plsc.VectorSubcoreMesh(core_axis_name="c", subcore_axis_name="s")
    kernel = pl.kernel(
        functools.partial(_scgather_body, blk, nblk, width),
        out_type=jax.ShapeDtypeStruct((n, width), table.dtype),
        mesh=mesh,
        scratch_types=[
            pltpu.VMEM((blk,), jnp.int32),
            pltpu.VMEM((blk, width), table.dtype),
            pltpu.SemaphoreType.DMA,
        ],
    )
    return kernel(table, idx)


def _fsel_body(nch, nc, bq, cand_ref, cid_ref, w_ref, idx_ref):
    iq = pl.program_id(0)
    rowg = iq * bq + lax.broadcasted_iota(jnp.int32, (bq, 1), 0)
    base = (cid_ref[...] - rowg * nch) * CH
    cidx = (base[:, :, None] +
            lax.broadcasted_iota(jnp.int32, (1, 1, CH), 2)).reshape(bq, nc)
    cur = cand_ref[...].astype(jnp.float32)
    kcol = lax.broadcasted_iota(jnp.int32, (bq, K), 1)
    scores = jnp.zeros((bq, K), jnp.float32)
    ids = jnp.zeros((bq, K), jnp.int32)
    for k in range(K):
        mx = jnp.max(cur, axis=1, keepdims=True)
        gi = jnp.min(jnp.where(cur == mx, cidx, BIG), axis=1, keepdims=True)
        scores = jnp.where(kcol == k, mx, scores)
        ids = jnp.where(kcol == k, gi, ids)
        cur = jnp.where(cidx == gi, NEG, cur)
    e = jnp.exp(scores - scores[:, 0:1])
    w_ref[...] = e / jnp.sum(e, axis=1, keepdims=True)
    idx_ref[...] = ids


def _final_select(cand, cid, nch):
    b = cid.shape[0]
    nc = K * CH
    bq = min(512, b)
    kernel = pl.pallas_call(
        functools.partial(_fsel_body, nch, nc, bq),
        grid=(b // bq,),
        in_specs=[
            pl.BlockSpec((bq, nc), lambda i: (i, 0)),
            pl.BlockSpec((bq, K), lambda i: (i, 0)),
        ],
        out_specs=[
            pl.BlockSpec((bq, K), lambda i: (i, 0)),
            pl.BlockSpec((bq, K), lambda i: (i, 0)),
        ],
        out_shape=[
            jax.ShapeDtypeStruct((b, K), jnp.float32),
            jax.ShapeDtypeStruct((b, K), jnp.int32),
        ],
    )
    return kernel(cand, cid)


def _wsum_body(rows_per_w, d, mv_hbm, idx_hbm, w_hbm, out_hbm, idx_sl, w_sl,
               vrows_ref, ret_sl, sem):
    wid = lax.axis_index("s") * 2 + lax.axis_index("c")
    dv = d // 16
    nk = rows_per_w * K
    base = wid * nk
    pltpu.sync_copy(idx_hbm.at[pl.ds(base, nk)], idx_sl)
    pltpu.sync_copy(w_hbm.at[pl.ds(base, nk)], w_sl.at[pl.ds(0, nk)])

    def row_body(i, _c):
        pltpu.async_copy(mv_hbm.at[idx_sl.at[pl.ds(i * K, K)]], vrows_ref,
                         sem).wait()

        def wsum(kk, acc):
            wk = w_sl[pl.ds(i * K + kk, 16)][0]
            return tuple(acc[j] + wk * vrows_ref[kk, pl.ds(j * 16, 16)]
                         for j in range(dv))

        acc = lax.fori_loop(
            0, K, wsum, tuple(jnp.zeros((16,), jnp.float32)
                              for _ in range(dv)))
        for j in range(dv):
            ret_sl[pl.ds(i * d + j * 16, 16)] = acc[j]
        return _c

    lax.fori_loop(0, rows_per_w, row_body, 0)
    pltpu.sync_copy(ret_sl, out_hbm.at[pl.ds(wid * rows_per_w * d,
                                             rows_per_w * d)])


def _sc_wsum(memory_values, idx, w):
    b = idx.shape[0]
    d = memory_values.shape[1]
    rows_per_w = b // 32
    mesh = plsc.VectorSubcoreMesh(core_axis_name="c", subcore_axis_name="s")
    kernel = pl.kernel(
        functools.partial(_wsum_body, rows_per_w, d),
        out_type=jax.ShapeDtypeStruct((b * d,), jnp.float32),
        mesh=mesh,
        scratch_types=[
            pltpu.VMEM((rows_per_w * K,), jnp.int32),
            pltpu.VMEM((rows_per_w * K + 16,), jnp.float32),
            pltpu.VMEM((K, d), jnp.float32),
            pltpu.VMEM((rows_per_w * d,), jnp.float32),
            pltpu.SemaphoreType.DMA,
        ],
    )
    out = kernel(memory_values, idx.reshape(-1), w.reshape(-1))
    return out.reshape(b, d)


def _ln_body(r_ref, q_ref, g_ref, b_ref, o_ref):
    x = r_ref[...] + q_ref[...]
    mu = jnp.mean(x, axis=1, keepdims=True)
    xc = x - mu
    var = jnp.mean(xc * xc, axis=1, keepdims=True)
    o_ref[...] = xc * lax.rsqrt(var + 1e-5) * g_ref[...] + b_ref[...]


def _layernorm(ret_raw, query, gamma, beta):
    b, d = query.shape
    bq = min(512, b)
    kernel = pl.pallas_call(
        _ln_body,
        grid=(b // bq,),
        in_specs=[
            pl.BlockSpec((bq, d), lambda i: (i, 0)),
            pl.BlockSpec((bq, d), lambda i: (i, 0)),
            pl.BlockSpec((1, d), lambda i: (0, 0)),
            pl.BlockSpec((1, d), lambda i: (0, 0)),
        ],
        out_specs=pl.BlockSpec((bq, d), lambda i: (i, 0)),
        out_shape=jax.ShapeDtypeStruct((b, d), jnp.float32),
    )
    return kernel(ret_raw, query, gamma.reshape(1, d), beta.reshape(1, d))


def kernel(query, memory_keys, memory_values, temperature, ln_gamma, ln_beta):
    b = query.shape[0]
    sims, cm, nch = _similarity(query, memory_keys, temperature)
    cid = _chunk_select(cm, nch)
    cand = _sc_gather(sims.reshape(-1, CH), cid.reshape(-1), CH)
    aw, idx = _final_select(cand.reshape(b, K * CH), cid, nch)
    ret_raw = _sc_wsum(memory_values, idx, aw)
    retrieved = _layernorm(ret_raw, query, ln_gamma, ln_beta)
    return retrieved, aw
